# Initial kernel scaffold; baseline (speedup 1.0000x reference)
import jax
import jax.numpy as jnp
from jax.experimental import pallas as pl


def kernel(pt_2d, color, pt_3d, normal, R, T, face):
    raise NotImplementedError("write your pallas kernel here")


# speedup vs baseline: 2.3978x; 2.3978x over previous
import functools

import jax
import jax.numpy as jnp
import numpy as np
from jax import lax
from jax.experimental import pallas as pl
from jax.experimental.pallas import tpu as pltpu
from jax.experimental.pallas import tpu_sc as plsc

FTINY = float(np.finfo(np.float32).tiny) * 1e3
INF_VALUE = float(np.finfo(np.float32).max) * 1e-3
LOWER_INF = float(np.finfo(np.float32).max) * 1e-4

H = 224
W = 224
NF = 4000
NV = 2100
FP = 4096
NVP = 2112
P = H * W

NC = 2
NS = 16
NWORK = NC * NS
L = 16

F_PER_W = FP // NWORK
PX_PER_W = P // NWORK
GS = 64
NPG = P // GS
NCHUNK = FP // 128


def _face_table_body(vtab_hbm, fidx_hbm, fnrm_hbm, geo_hbm, ctab_hbm,
                     vtab_v, fidx_v, fnrm_v, geo_v, ctab_loc):
    wid = lax.axis_index("s") * NC + lax.axis_index("c")
    base = wid * F_PER_W
    pltpu.sync_copy(vtab_hbm, vtab_v)
    for k in range(3):
        pltpu.sync_copy(fidx_hbm.at[pl.ds(k * FP + base, F_PER_W)],
                        fidx_v.at[pl.ds(k * F_PER_W, F_PER_W)])
        pltpu.sync_copy(fnrm_hbm.at[pl.ds(k * FP + base, F_PER_W)],
                        fnrm_v.at[pl.ds(k * F_PER_W, F_PER_W)])

    for j in range(F_PER_W // L):
        i0 = fidx_v[pl.ds(j * L, L)]
        i1 = fidx_v[pl.ds(F_PER_W + j * L, L)]
        i2 = fidx_v[pl.ds(2 * F_PER_W + j * L, L)]

        def ga(row, idx):
            return plsc.load_gather(vtab_v, [idx + row * NVP])

        x0, x1, x2 = ga(0, i0), ga(0, i1), ga(0, i2)
        y0, y1, y2 = ga(1, i0), ga(1, i1), ga(1, i2)
        z0, z1, z2 = ga(2, i0), ga(2, i1), ga(2, i2)
        ndot = (ga(6, i0) * fnrm_v[pl.ds(j * L, L)]
                + ga(7, i0) * fnrm_v[pl.ds(F_PER_W + j * L, L)]
                + ga(8, i0) * fnrm_v[pl.ds(2 * F_PER_W + j * L, L)])
        valid = (ndot < 0.0) & (jnp.minimum(z0, jnp.minimum(z1, z2)) > 0.0)

        det = (y1 - y2) * (x0 - x2) + (x2 - x1) * (y0 - y2)
        det = jnp.sign(det) * jnp.maximum(jnp.abs(det), FTINY)
        inv = 1.0 / det
        l0x = (y1 - y2) * inv
        l0y = (x2 - x1) * inv
        l0c = -l0x * x2 - l0y * y2
        l1x = (y2 - y0) * inv
        l1y = (x0 - x2) * inv
        l1c = -l1x * x2 - l1y * y2
        l2x = -l0x - l1x
        l2y = -l0y - l1y
        l2c = 1.0 - l0c - l1c
        dx = z0 * l0x + z1 * l1x + z2 * l2x
        dy = z0 * l0y + z1 * l1y + z2 * l2y
        dc = z0 * l0c + z1 * l1c + z2 * l2c
        l0c_eff = jnp.where(valid, l0c, jnp.float32(-jnp.inf))

        geo = [l0x, l0y, l0c_eff, l1x, l1y, l1c, l2x, l2y, l2c, dx, dy, dc]
        for k, v in enumerate(geo):
            geo_v[pl.ds(k * F_PER_W + j * L, L)] = v
        for n, crow in enumerate((3, 4, 5)):
            c0, c1, c2 = ga(crow, i0), ga(crow, i1), ga(crow, i2)
            cx = c0 * l0x + c1 * l1x + c2 * l2x
            cy = c0 * l0y + c1 * l1y + c2 * l2y
            cc = c0 * l0c + c1 * l1c + c2 * l2c
            ctab_loc[pl.ds((3 * n) * F_PER_W + j * L, L)] = cx
            ctab_loc[pl.ds((3 * n + 1) * F_PER_W + j * L, L)] = cy
            ctab_loc[pl.ds((3 * n + 2) * F_PER_W + j * L, L)] = cc

    for k in range(12):
        pltpu.sync_copy(geo_v.at[pl.ds(k * F_PER_W, F_PER_W)],
                        geo_hbm.at[pl.ds(k * FP + base, F_PER_W)])
    for k in range(9):
        pltpu.sync_copy(ctab_loc.at[pl.ds(k * F_PER_W, F_PER_W)],
                        ctab_hbm.at[pl.ds(k * FP + base, F_PER_W)])


def _face_table(vtab, fidx, fnrm):
    mesh = plsc.VectorSubcoreMesh(core_axis_name="c", subcore_axis_name="s")
    kfn = functools.partial(
        pl.kernel,
        out_type=[
            jax.ShapeDtypeStruct((12 * FP,), jnp.float32),
            jax.ShapeDtypeStruct((9 * FP,), jnp.float32),
        ],
        mesh=mesh,
        scratch_types=[
            pltpu.VMEM((9 * NVP,), jnp.float32),
            pltpu.VMEM((3 * F_PER_W,), jnp.int32),
            pltpu.VMEM((3 * F_PER_W,), jnp.float32),
            pltpu.VMEM((12 * F_PER_W,), jnp.float32),
            pltpu.VMEM((9 * F_PER_W,), jnp.float32),
        ],
        compiler_params=pltpu.CompilerParams(needs_layout_passes=False),
    )(_face_table_body)
    return kfn(vtab, fidx, fnrm)


def _zbuf_body(geo_ref, mask_ref, idx_ref):
    lane = lax.broadcasted_iota(jnp.int32, (GS, 128), 1)

    def pg_body(pg, _):
        off = pl.multiple_of(pg * GS, GS)
        p = off + lax.broadcasted_iota(jnp.int32, (GS, 1), 0)
        yi = p // W
        xv = (p - yi * W).astype(jnp.float32)
        yv = yi.astype(jnp.float32)

        def chunk_body(c, carry):
            best, bidx = carry
            cs = pl.ds(pl.multiple_of(c * 128, 128), 128)

            def t(k):
                return geo_ref[k:k + 1, cs]

            l0 = xv * t(0) + yv * t(1) + t(2)
            l1 = xv * t(3) + yv * t(4) + t(5)
            l2 = xv * t(6) + yv * t(7) + t(8)
            dd = xv * t(9) + yv * t(10) + t(11)
            m = (l0 >= 0.0) & (l1 >= 0.0) & (l2 >= 0.0)
            dd = dd + jnp.where(m, 0.0, INF_VALUE)
            dd = jnp.where(dd != dd, INF_VALUE, dd)
            upd = dd < best
            best = jnp.where(upd, dd, best)
            bidx = jnp.where(upd, lane + c * 128, bidx)
            return best, bidx

        best = jnp.full((GS, 128), INF_VALUE, jnp.float32)
        bidx = jnp.zeros((GS, 128), jnp.int32)
        best, bidx = lax.fori_loop(0, NCHUNK, chunk_body, (best, bidx))
        gm = jnp.min(best, axis=1, keepdims=True)
        eq = best == gm
        cand = jnp.where(eq, bidx, jnp.int32(2 ** 30))
        arg = jnp.min(cand, axis=1, keepdims=True)
        vis = gm < LOWER_INF
        mask_ref[pl.ds(off, GS), 0:1] = vis.astype(jnp.float32)
        idx_ref[pl.ds(off, GS), 0:1] = jnp.where(vis, arg, 0)
        return 0

    lax.fori_loop(0, NPG, pg_body, 0)


def _zbuffer(geo):
    return pl.pallas_call(
        _zbuf_body,
        out_shape=[
            jax.ShapeDtypeStruct((P, 8), jnp.float32),
            jax.ShapeDtypeStruct((P, 8), jnp.int32),
        ],
    )(geo)


def _compose_body(ctab_hbm, idx_hbm, mask_hbm, xg_hbm, yg_hbm, img_hbm,
                  ctab_v, idx_v, mask_v, xg_v, yg_v, out_v):
    wid = lax.axis_index("s") * NC + lax.axis_index("c")
    base = wid * PX_PER_W
    pltpu.sync_copy(ctab_hbm, ctab_v)
    pltpu.sync_copy(idx_hbm.at[pl.ds(base, PX_PER_W)], idx_v)
    pltpu.sync_copy(mask_hbm.at[pl.ds(base, PX_PER_W)], mask_v)
    pltpu.sync_copy(xg_hbm.at[pl.ds(base, PX_PER_W)], xg_v)
    pltpu.sync_copy(yg_hbm.at[pl.ds(base, PX_PER_W)], yg_v)

    for j in range(PX_PER_W // L):
        sl = pl.ds(j * L, L)
        iv = idx_v[sl]
        mv = mask_v[sl]
        xv = xg_v[sl]
        yv = yg_v[sl]
        for c in range(3):
            cx = plsc.load_gather(ctab_v, [iv + (3 * c) * FP])
            cy = plsc.load_gather(ctab_v, [iv + (3 * c + 1) * FP])
            cc = plsc.load_gather(ctab_v, [iv + (3 * c + 2) * FP])
            out_v[pl.ds(c * PX_PER_W + j * L, L)] = mv * (cx * xv + cy * yv + cc)

    for c in range(3):
        pltpu.sync_copy(out_v.at[pl.ds(c * PX_PER_W, PX_PER_W)],
                        img_hbm.at[pl.ds(c * P + base, PX_PER_W)])


def _compose(ctab, idx, mask, xg, yg):
    mesh = plsc.VectorSubcoreMesh(core_axis_name="c", subcore_axis_name="s")
    kfn = functools.partial(
        pl.kernel,
        out_type=jax.ShapeDtypeStruct((3 * P,), jnp.float32),
        mesh=mesh,
        scratch_types=[
            pltpu.VMEM((9 * FP,), jnp.float32),
            pltpu.VMEM((PX_PER_W,), jnp.int32),
            pltpu.VMEM((PX_PER_W,), jnp.float32),
            pltpu.VMEM((PX_PER_W,), jnp.float32),
            pltpu.VMEM((PX_PER_W,), jnp.float32),
            pltpu.VMEM((3 * PX_PER_W,), jnp.float32),
        ],
        compiler_params=pltpu.CompilerParams(needs_layout_passes=False),
    )(_compose_body)
    return kfn(ctab, idx, mask, xg, yg)


def kernel(pt_2d, color, pt_3d, normal, R, T, face):
    q = pt_3d[0] + R[0].T @ T[0]
    vtab = jnp.concatenate([pt_2d[0], color[0], q], axis=0)
    vtab = jnp.pad(vtab, ((0, 0), (0, NVP - NV))).reshape(-1)
    fidx = jnp.pad(face, ((0, 0), (0, FP - NF))).reshape(-1)
    fnrm = jnp.pad(normal[0], ((0, 0), (0, FP - NF))).reshape(-1)

    geo_flat, ctab = _face_table(vtab, fidx, fnrm)
    geo = geo_flat.reshape(12, FP)

    mask8, idx8 = _zbuffer(geo)
    mask = mask8[:, 0]
    idx = idx8[:, 0]

    xg = jnp.tile(jnp.arange(W, dtype=jnp.float32), H)
    yg = jnp.repeat(jnp.arange(H, dtype=jnp.float32), W)
    img = _compose(ctab, idx, mask, xg, yg)
    image = img.reshape(1, 3, H, W)
    return image, mask.reshape(1, H, W)


# speedup vs baseline: 2.7844x; 1.1612x over previous
import functools

import jax
import jax.numpy as jnp
import numpy as np
from jax import lax
from jax.experimental import pallas as pl
from jax.experimental.pallas import tpu as pltpu
from jax.experimental.pallas import tpu_sc as plsc

FTINY = float(np.finfo(np.float32).tiny) * 1e3
INF_VALUE = float(np.finfo(np.float32).max) * 1e-3
LOWER_INF = float(np.finfo(np.float32).max) * 1e-4

H = 224
W = 224
NF = 4000
NV = 2100
FP = 4096
NVP = 2112
P = H * W

NC = 2
NS = 16
NWORK = NC * NS
L = 16

F_PER_W = FP // NWORK
PX_PER_W = P // NWORK
GS = 64
NPG = P // GS
NCHUNK = FP // 128


def _face_table_body(vtab_hbm, fidx_hbm, fnrm_hbm, geo_hbm, ctab_hbm,
                     vtab_v, fidx_v, fnrm_v, geo_v, ctab_loc):
    wid = lax.axis_index("s") * NC + lax.axis_index("c")
    base = wid * F_PER_W
    pltpu.sync_copy(vtab_hbm, vtab_v)
    for k in range(3):
        pltpu.sync_copy(fidx_hbm.at[pl.ds(k * FP + base, F_PER_W)],
                        fidx_v.at[pl.ds(k * F_PER_W, F_PER_W)])
        pltpu.sync_copy(fnrm_hbm.at[pl.ds(k * FP + base, F_PER_W)],
                        fnrm_v.at[pl.ds(k * F_PER_W, F_PER_W)])

    for j in range(F_PER_W // L):
        i0 = fidx_v[pl.ds(j * L, L)]
        i1 = fidx_v[pl.ds(F_PER_W + j * L, L)]
        i2 = fidx_v[pl.ds(2 * F_PER_W + j * L, L)]

        def ga(row, idx):
            return plsc.load_gather(vtab_v, [idx + row * NVP])

        x0, x1, x2 = ga(0, i0), ga(0, i1), ga(0, i2)
        y0, y1, y2 = ga(1, i0), ga(1, i1), ga(1, i2)
        z0, z1, z2 = ga(2, i0), ga(2, i1), ga(2, i2)
        ndot = (ga(6, i0) * fnrm_v[pl.ds(j * L, L)]
                + ga(7, i0) * fnrm_v[pl.ds(F_PER_W + j * L, L)]
                + ga(8, i0) * fnrm_v[pl.ds(2 * F_PER_W + j * L, L)])
        valid = (ndot < 0.0) & (jnp.minimum(z0, jnp.minimum(z1, z2)) > 0.0)

        det = (y1 - y2) * (x0 - x2) + (x2 - x1) * (y0 - y2)
        det = jnp.sign(det) * jnp.maximum(jnp.abs(det), FTINY)
        inv = 1.0 / det
        l0x = (y1 - y2) * inv
        l0y = (x2 - x1) * inv
        l0c = -l0x * x2 - l0y * y2
        l1x = (y2 - y0) * inv
        l1y = (x0 - x2) * inv
        l1c = -l1x * x2 - l1y * y2
        l2x = -l0x - l1x
        l2y = -l0y - l1y
        l2c = 1.0 - l0c - l1c
        dx = z0 * l0x + z1 * l1x + z2 * l2x
        dy = z0 * l0y + z1 * l1y + z2 * l2y
        dc = z0 * l0c + z1 * l1c + z2 * l2c
        l0c_eff = jnp.where(valid, l0c, jnp.float32(-jnp.inf))

        geo = [l0x, l0y, l0c_eff, l1x, l1y, l1c, l2x, l2y, l2c, dx, dy, dc]
        for k, v in enumerate(geo):
            geo_v[pl.ds(k * F_PER_W + j * L, L)] = v
        for n, crow in enumerate((3, 4, 5)):
            c0, c1, c2 = ga(crow, i0), ga(crow, i1), ga(crow, i2)
            cx = c0 * l0x + c1 * l1x + c2 * l2x
            cy = c0 * l0y + c1 * l1y + c2 * l2y
            cc = c0 * l0c + c1 * l1c + c2 * l2c
            ctab_loc[pl.ds((3 * n) * F_PER_W + j * L, L)] = cx
            ctab_loc[pl.ds((3 * n + 1) * F_PER_W + j * L, L)] = cy
            ctab_loc[pl.ds((3 * n + 2) * F_PER_W + j * L, L)] = cc

    for k in range(12):
        pltpu.sync_copy(geo_v.at[pl.ds(k * F_PER_W, F_PER_W)],
                        geo_hbm.at[pl.ds(k * FP + base, F_PER_W)])
    for k in range(9):
        pltpu.sync_copy(ctab_loc.at[pl.ds(k * F_PER_W, F_PER_W)],
                        ctab_hbm.at[pl.ds(k * FP + base, F_PER_W)])


def _face_table(vtab, fidx, fnrm):
    mesh = plsc.VectorSubcoreMesh(core_axis_name="c", subcore_axis_name="s")
    kfn = functools.partial(
        pl.kernel,
        out_type=[
            jax.ShapeDtypeStruct((12 * FP,), jnp.float32),
            jax.ShapeDtypeStruct((9 * FP,), jnp.float32),
        ],
        mesh=mesh,
        scratch_types=[
            pltpu.VMEM((9 * NVP,), jnp.float32),
            pltpu.VMEM((3 * F_PER_W,), jnp.int32),
            pltpu.VMEM((3 * F_PER_W,), jnp.float32),
            pltpu.VMEM((12 * F_PER_W,), jnp.float32),
            pltpu.VMEM((9 * F_PER_W,), jnp.float32),
        ],
        compiler_params=pltpu.CompilerParams(needs_layout_passes=False),
    )(_face_table_body)
    return kfn(vtab, fidx, fnrm)


def _zbuf_body(geo_ref, mask_ref, idx_ref):
    QS = 56
    xv0 = lax.broadcasted_iota(jnp.int32, (QS, 128), 0).astype(jnp.float32)
    lane = lax.broadcasted_iota(jnp.int32, (QS, 128), 1)

    def row_body(rq, _):
        r = rq // 4
        qq = rq - r * 4
        yf = r.astype(jnp.float32)
        xv = xv0 + (qq * QS).astype(jnp.float32)

        def chunk_body(c, carry):
            best, bidx = carry
            cs = pl.ds(pl.multiple_of(c * 128, 128), 128)

            def t(k):
                return geo_ref[k:k + 1, cs]

            l0 = xv * t(0) + (yf * t(1) + t(2))
            l1 = xv * t(3) + (yf * t(4) + t(5))
            l2 = xv * t(6) + (yf * t(7) + t(8))
            m = jnp.minimum(jnp.minimum(l0, l1), l2) >= 0.0
            dd = xv * t(9) + (yf * t(10) + t(11))
            dd = dd + jnp.where(m, 0.0, INF_VALUE)
            dd = jnp.where(dd != dd, INF_VALUE, dd)
            upd = dd < best
            best = jnp.minimum(best, dd)
            bidx = jnp.where(upd, lane + c * 128, bidx)
            return best, bidx

        best = jnp.full((QS, 128), INF_VALUE, jnp.float32)
        bidx = jnp.zeros((QS, 128), jnp.int32)
        best, bidx = lax.fori_loop(0, NCHUNK, chunk_body, (best, bidx))
        gm = jnp.min(best, axis=1, keepdims=True)
        eq = best == gm
        cand = jnp.where(eq, bidx, jnp.int32(2 ** 30))
        arg = jnp.min(cand, axis=1, keepdims=True)
        vis = gm < LOWER_INF
        off = pl.multiple_of(rq * QS, 8)
        mask_ref[pl.ds(off, QS), 0:1] = vis.astype(jnp.float32)
        idx_ref[pl.ds(off, QS), 0:1] = jnp.where(vis, arg, 0)
        return 0

    lax.fori_loop(0, H * 4, row_body, 0)


def _zbuffer(geo):
    return pl.pallas_call(
        _zbuf_body,
        out_shape=[
            jax.ShapeDtypeStruct((P, 8), jnp.float32),
            jax.ShapeDtypeStruct((P, 8), jnp.int32),
        ],
    )(geo)


def _compose_body(ctab_hbm, idx_hbm, mask_hbm, xg_hbm, yg_hbm, img_hbm,
                  ctab_v, idx_v, mask_v, xg_v, yg_v, out_v):
    wid = lax.axis_index("s") * NC + lax.axis_index("c")
    base = wid * PX_PER_W
    pltpu.sync_copy(ctab_hbm, ctab_v)
    pltpu.sync_copy(idx_hbm.at[pl.ds(base, PX_PER_W)], idx_v)
    pltpu.sync_copy(mask_hbm.at[pl.ds(base, PX_PER_W)], mask_v)
    pltpu.sync_copy(xg_hbm.at[pl.ds(base, PX_PER_W)], xg_v)
    pltpu.sync_copy(yg_hbm.at[pl.ds(base, PX_PER_W)], yg_v)

    for j in range(PX_PER_W // L):
        sl = pl.ds(j * L, L)
        iv = idx_v[sl]
        mv = mask_v[sl]
        xv = xg_v[sl]
        yv = yg_v[sl]
        for c in range(3):
            cx = plsc.load_gather(ctab_v, [iv + (3 * c) * FP])
            cy = plsc.load_gather(ctab_v, [iv + (3 * c + 1) * FP])
            cc = plsc.load_gather(ctab_v, [iv + (3 * c + 2) * FP])
            out_v[pl.ds(c * PX_PER_W + j * L, L)] = mv * (cx * xv + cy * yv + cc)

    for c in range(3):
        pltpu.sync_copy(out_v.at[pl.ds(c * PX_PER_W, PX_PER_W)],
                        img_hbm.at[pl.ds(c * P + base, PX_PER_W)])


def _compose(ctab, idx, mask, xg, yg):
    mesh = plsc.VectorSubcoreMesh(core_axis_name="c", subcore_axis_name="s")
    kfn = functools.partial(
        pl.kernel,
        out_type=jax.ShapeDtypeStruct((3 * P,), jnp.float32),
        mesh=mesh,
        scratch_types=[
            pltpu.VMEM((9 * FP,), jnp.float32),
            pltpu.VMEM((PX_PER_W,), jnp.int32),
            pltpu.VMEM((PX_PER_W,), jnp.float32),
            pltpu.VMEM((PX_PER_W,), jnp.float32),
            pltpu.VMEM((PX_PER_W,), jnp.float32),
            pltpu.VMEM((3 * PX_PER_W,), jnp.float32),
        ],
        compiler_params=pltpu.CompilerParams(needs_layout_passes=False),
    )(_compose_body)
    return kfn(ctab, idx, mask, xg, yg)


def kernel(pt_2d, color, pt_3d, normal, R, T, face):
    q = pt_3d[0] + R[0].T @ T[0]
    vtab = jnp.concatenate([pt_2d[0], color[0], q], axis=0)
    vtab = jnp.pad(vtab, ((0, 0), (0, NVP - NV))).reshape(-1)
    fidx = jnp.pad(face, ((0, 0), (0, FP - NF))).reshape(-1)
    fnrm = jnp.pad(normal[0], ((0, 0), (0, FP - NF))).reshape(-1)

    geo_flat, ctab = _face_table(vtab, fidx, fnrm)
    geo = geo_flat.reshape(12, FP)

    mask8, idx8 = _zbuffer(geo)
    mask = mask8[:, 0]
    idx = idx8[:, 0]

    xg = jnp.tile(jnp.arange(W, dtype=jnp.float32), H)
    yg = jnp.repeat(jnp.arange(H, dtype=jnp.float32), W)
    img = _compose(ctab, idx, mask, xg, yg)
    image = img.reshape(1, 3, H, W)
    return image, mask.reshape(1, H, W)


# speedup vs baseline: 4.1000x; 1.4725x over previous
import functools

import jax
import jax.numpy as jnp
import numpy as np
from jax import lax
from jax.experimental import pallas as pl
from jax.experimental.pallas import tpu as pltpu
from jax.experimental.pallas import tpu_sc as plsc

FTINY = float(np.finfo(np.float32).tiny) * 1e3
INF_VALUE = float(np.finfo(np.float32).max) * 1e-3
LOWER_INF = float(np.finfo(np.float32).max) * 1e-4
CULL_TH = 2.8e35

H = 224
W = 224
NF = 4000
NV = 2100
FP = 4096
NVP = 2112
P = H * W

NC = 2
NS = 16
NWORK = NC * NS
L = 16

F_PER_W = FP // NWORK
PX_PER_W = P // NWORK
NCHUNK = FP // 128


def _face_table_body(vtab_hbm, fidx_hbm, fnrm_hbm, geo_hbm, ctab_hbm, keep_hbm,
                     vtab_v, fidx_v, fnrm_v, geo_v, ctab_loc, keep_v):
    wid = lax.axis_index("s") * NC + lax.axis_index("c")
    base = wid * F_PER_W
    pltpu.sync_copy(vtab_hbm, vtab_v)
    for k in range(3):
        pltpu.sync_copy(fidx_hbm.at[pl.ds(k * FP + base, F_PER_W)],
                        fidx_v.at[pl.ds(k * F_PER_W, F_PER_W)])
        pltpu.sync_copy(fnrm_hbm.at[pl.ds(k * FP + base, F_PER_W)],
                        fnrm_v.at[pl.ds(k * F_PER_W, F_PER_W)])

    lane = lax.iota(jnp.int32, L)
    for j in range(F_PER_W // L):
        i0 = fidx_v[pl.ds(j * L, L)]
        i1 = fidx_v[pl.ds(F_PER_W + j * L, L)]
        i2 = fidx_v[pl.ds(2 * F_PER_W + j * L, L)]

        def ga(row, idx):
            return plsc.load_gather(vtab_v, [idx + row * NVP])

        x0, x1, x2 = ga(0, i0), ga(0, i1), ga(0, i2)
        y0, y1, y2 = ga(1, i0), ga(1, i1), ga(1, i2)
        z0, z1, z2 = ga(2, i0), ga(2, i1), ga(2, i2)
        ndot = (ga(6, i0) * fnrm_v[pl.ds(j * L, L)]
                + ga(7, i0) * fnrm_v[pl.ds(F_PER_W + j * L, L)]
                + ga(8, i0) * fnrm_v[pl.ds(2 * F_PER_W + j * L, L)])
        valid = (ndot < 0.0) & (jnp.minimum(z0, jnp.minimum(z1, z2)) > 0.0)

        det = (y1 - y2) * (x0 - x2) + (x2 - x1) * (y0 - y2)
        det = jnp.sign(det) * jnp.maximum(jnp.abs(det), FTINY)
        inv = 1.0 / det
        l0x = (y1 - y2) * inv
        l0y = (x2 - x1) * inv
        l0c = -l0x * x2 - l0y * y2
        l1x = (y2 - y0) * inv
        l1y = (x0 - x2) * inv
        l1c = -l1x * x2 - l1y * y2
        l2x = -l0x - l1x
        l2y = -l0y - l1y
        l2c = 1.0 - l0c - l1c
        dx = z0 * l0x + z1 * l1x + z2 * l2x
        dy = z0 * l0y + z1 * l1y + z2 * l2y
        dc = z0 * l0c + z1 * l1c + z2 * l2c
        l0c_eff = jnp.where(valid, l0c, jnp.float32(-jnp.inf))

        geo = [l0x, l0y, l0c_eff, l1x, l1y, l1c, l2x, l2y, l2c, dx, dy, dc]
        for k, v in enumerate(geo):
            geo_v[pl.ds(k * F_PER_W + j * L, L)] = v
        for n, crow in enumerate((3, 4, 5)):
            c0, c1, c2 = ga(crow, i0), ga(crow, i1), ga(crow, i2)
            cx = c0 * l0x + c1 * l1x + c2 * l2x
            cy = c0 * l0y + c1 * l1y + c2 * l2y
            cc = c0 * l0c + c1 * l1c + c2 * l2c
            ctab_loc[pl.ds((3 * n) * F_PER_W + j * L, L)] = cx
            ctab_loc[pl.ds((3 * n + 1) * F_PER_W + j * L, L)] = cy
            ctab_loc[pl.ds((3 * n + 2) * F_PER_W + j * L, L)] = cc

        bound = (jnp.abs(dx) + jnp.abs(dy)) * 224.0 + jnp.abs(dc)
        danger = (bound >= CULL_TH) | (bound != bound)
        gid = base + j * L + lane
        keep = (valid | danger) & (gid < NF)
        keep_v[pl.ds(j * L, L)] = keep.astype(jnp.int32)

    for k in range(12):
        pltpu.sync_copy(geo_v.at[pl.ds(k * F_PER_W, F_PER_W)],
                        geo_hbm.at[pl.ds(k * FP + base, F_PER_W)])
    for k in range(9):
        pltpu.sync_copy(ctab_loc.at[pl.ds(k * F_PER_W, F_PER_W)],
                        ctab_hbm.at[pl.ds(k * FP + base, F_PER_W)])
    pltpu.sync_copy(keep_v, keep_hbm.at[pl.ds(base, F_PER_W)])


def _face_table(vtab, fidx, fnrm):
    mesh = plsc.VectorSubcoreMesh(core_axis_name="c", subcore_axis_name="s")
    kfn = functools.partial(
        pl.kernel,
        out_type=[
            jax.ShapeDtypeStruct((12 * FP,), jnp.float32),
            jax.ShapeDtypeStruct((9 * FP,), jnp.float32),
            jax.ShapeDtypeStruct((FP,), jnp.int32),
        ],
        mesh=mesh,
        scratch_types=[
            pltpu.VMEM((9 * NVP,), jnp.float32),
            pltpu.VMEM((3 * F_PER_W,), jnp.int32),
            pltpu.VMEM((3 * F_PER_W,), jnp.float32),
            pltpu.VMEM((12 * F_PER_W,), jnp.float32),
            pltpu.VMEM((9 * F_PER_W,), jnp.float32),
            pltpu.VMEM((F_PER_W,), jnp.int32),
        ],
        compiler_params=pltpu.CompilerParams(needs_layout_passes=False),
    )(_face_table_body)
    return kfn(vtab, fidx, fnrm)


def _compact_body(geo_hbm, perm_hbm, geoc_hbm, geo_v, perm_v, out_v):
    wid = lax.axis_index("s") * NC + lax.axis_index("c")
    base = wid * F_PER_W
    pltpu.sync_copy(geo_hbm, geo_v)
    pltpu.sync_copy(perm_hbm.at[pl.ds(base, F_PER_W)], perm_v)

    for j in range(F_PER_W // L):
        pv = perm_v[pl.ds(j * L, L)]
        for k in range(12):
            g = plsc.load_gather(geo_v, [pv + k * FP])
            out_v[pl.ds(k * F_PER_W + j * L, L)] = g

    for k in range(12):
        pltpu.sync_copy(out_v.at[pl.ds(k * F_PER_W, F_PER_W)],
                        geoc_hbm.at[pl.ds(k * FP + base, F_PER_W)])


def _compact(geo, perm):
    mesh = plsc.VectorSubcoreMesh(core_axis_name="c", subcore_axis_name="s")
    kfn = functools.partial(
        pl.kernel,
        out_type=jax.ShapeDtypeStruct((12 * FP,), jnp.float32),
        mesh=mesh,
        scratch_types=[
            pltpu.VMEM((12 * FP,), jnp.float32),
            pltpu.VMEM((F_PER_W,), jnp.int32),
            pltpu.VMEM((12 * F_PER_W,), jnp.float32),
        ],
        compiler_params=pltpu.CompilerParams(needs_layout_passes=False),
    )(_compact_body)
    return kfn(geo, perm)


def _zbuf_body(nc_ref, geo_ref, perm_ref, mask_ref, idx_ref):
    QS = 56
    xv0 = lax.broadcasted_iota(jnp.int32, (QS, 128), 0).astype(jnp.float32)
    nchunk = nc_ref[0]

    def row_body(rq, _):
        r = rq // 4
        qq = rq - r * 4
        yf = r.astype(jnp.float32)
        xv = xv0 + (qq * QS).astype(jnp.float32)

        def chunk_body(c, carry):
            best, bidx = carry
            cs = pl.ds(pl.multiple_of(c * 128, 128), 128)

            def t(k):
                return geo_ref[k:k + 1, cs]

            ids = perm_ref[0:1, cs]
            l0 = xv * t(0) + (yf * t(1) + t(2))
            l1 = xv * t(3) + (yf * t(4) + t(5))
            l2 = xv * t(6) + (yf * t(7) + t(8))
            m = jnp.minimum(jnp.minimum(l0, l1), l2) >= 0.0
            dd = xv * t(9) + (yf * t(10) + t(11))
            dd = dd + jnp.where(m, 0.0, INF_VALUE)
            dd = jnp.where(dd != dd, INF_VALUE, dd)
            upd = dd < best
            best = jnp.minimum(best, dd)
            bidx = jnp.where(upd, jnp.broadcast_to(ids, (QS, 128)), bidx)
            return best, bidx

        best = jnp.full((QS, 128), INF_VALUE, jnp.float32)
        bidx = jnp.zeros((QS, 128), jnp.int32)
        best, bidx = lax.fori_loop(0, nchunk, chunk_body, (best, bidx))
        gm = jnp.min(best, axis=1, keepdims=True)
        eq = best == gm
        cand = jnp.where(eq, bidx, jnp.int32(2 ** 30))
        arg = jnp.min(cand, axis=1, keepdims=True)
        vis = gm < LOWER_INF
        off = pl.multiple_of(rq * QS, 8)
        mask_ref[pl.ds(off, QS), 0:1] = vis.astype(jnp.float32)
        idx_ref[pl.ds(off, QS), 0:1] = jnp.where(vis, arg, 0)
        return 0

    lax.fori_loop(0, H * 4, row_body, 0)


def _zbuffer(nchunks, geo, perm):
    return pl.pallas_call(
        _zbuf_body,
        in_specs=[
            pl.BlockSpec(memory_space=pltpu.SMEM),
            pl.BlockSpec(memory_space=pltpu.VMEM),
            pl.BlockSpec(memory_space=pltpu.VMEM),
        ],
        out_shape=[
            jax.ShapeDtypeStruct((P, 8), jnp.float32),
            jax.ShapeDtypeStruct((P, 8), jnp.int32),
        ],
    )(nchunks, geo, perm)


def _compose_body(ctab_hbm, idx_hbm, mask_hbm, xg_hbm, yg_hbm, img_hbm,
                  ctab_v, idx_v, mask_v, xg_v, yg_v, out_v):
    wid = lax.axis_index("s") * NC + lax.axis_index("c")
    base = wid * PX_PER_W
    pltpu.sync_copy(ctab_hbm, ctab_v)
    pltpu.sync_copy(idx_hbm.at[pl.ds(base, PX_PER_W)], idx_v)
    pltpu.sync_copy(mask_hbm.at[pl.ds(base, PX_PER_W)], mask_v)
    pltpu.sync_copy(xg_hbm.at[pl.ds(base, PX_PER_W)], xg_v)
    pltpu.sync_copy(yg_hbm.at[pl.ds(base, PX_PER_W)], yg_v)

    for j in range(PX_PER_W // L):
        sl = pl.ds(j * L, L)
        iv = idx_v[sl]
        mv = mask_v[sl]
        xv = xg_v[sl]
        yv = yg_v[sl]
        for c in range(3):
            cx = plsc.load_gather(ctab_v, [iv + (3 * c) * FP])
            cy = plsc.load_gather(ctab_v, [iv + (3 * c + 1) * FP])
            cc = plsc.load_gather(ctab_v, [iv + (3 * c + 2) * FP])
            out_v[pl.ds(c * PX_PER_W + j * L, L)] = mv * (cx * xv + cy * yv + cc)

    for c in range(3):
        pltpu.sync_copy(out_v.at[pl.ds(c * PX_PER_W, PX_PER_W)],
                        img_hbm.at[pl.ds(c * P + base, PX_PER_W)])


def _compose(ctab, idx, mask, xg, yg):
    mesh = plsc.VectorSubcoreMesh(core_axis_name="c", subcore_axis_name="s")
    kfn = functools.partial(
        pl.kernel,
        out_type=jax.ShapeDtypeStruct((3 * P,), jnp.float32),
        mesh=mesh,
        scratch_types=[
            pltpu.VMEM((9 * FP,), jnp.float32),
            pltpu.VMEM((PX_PER_W,), jnp.int32),
            pltpu.VMEM((PX_PER_W,), jnp.float32),
            pltpu.VMEM((PX_PER_W,), jnp.float32),
            pltpu.VMEM((PX_PER_W,), jnp.float32),
            pltpu.VMEM((3 * PX_PER_W,), jnp.float32),
        ],
        compiler_params=pltpu.CompilerParams(needs_layout_passes=False),
    )(_compose_body)
    return kfn(ctab, idx, mask, xg, yg)


def kernel(pt_2d, color, pt_3d, normal, R, T, face):
    q = pt_3d[0] + R[0].T @ T[0]
    vtab = jnp.concatenate([pt_2d[0], color[0], q], axis=0)
    vtab = jnp.pad(vtab, ((0, 0), (0, NVP - NV))).reshape(-1)
    fidx = jnp.pad(face, ((0, 0), (0, FP - NF))).reshape(-1)
    fnrm = jnp.pad(normal[0], ((0, 0), (0, FP - NF))).reshape(-1)

    geo_flat, ctab, keep = _face_table(vtab, fidx, fnrm)

    perm = jnp.nonzero(keep, size=FP, fill_value=NF)[0].astype(jnp.int32)
    nkeep = jnp.sum(keep)
    nchunks = ((nkeep + 127) // 128).astype(jnp.int32).reshape(1)

    geoc = _compact(geo_flat, perm).reshape(12, FP)
    mask8, idx8 = _zbuffer(nchunks, geoc, perm.reshape(1, FP))
    mask = mask8[:, 0]
    idx = idx8[:, 0]

    xg = jnp.tile(jnp.arange(W, dtype=jnp.float32), H)
    yg = jnp.repeat(jnp.arange(H, dtype=jnp.float32), W)
    img = _compose(ctab, idx, mask, xg, yg)
    image = img.reshape(1, 3, H, W)
    return image, mask.reshape(1, H, W)


# speedup vs baseline: 5.2952x; 1.2915x over previous
import functools

import jax
import jax.numpy as jnp
import numpy as np
from jax import lax
from jax.experimental import pallas as pl
from jax.experimental.pallas import tpu as pltpu
from jax.experimental.pallas import tpu_sc as plsc

FTINY = float(np.finfo(np.float32).tiny) * 1e3
INF_VALUE = float(np.finfo(np.float32).max) * 1e-3
LOWER_INF = float(np.finfo(np.float32).max) * 1e-4
CULL_TH = 2.8e35

H = 224
W = 224
NF = 4000
NV = 2100
FP = 4096
NVP = 2112
P = H * W

NC = 2
NS = 16
NWORK = NC * NS
L = 16

F_PER_W = FP // NWORK
PX_PER_W = P // NWORK
NCHUNK = FP // 128


def _face_table_body(vtab_hbm, fidx_hbm, fnrm_hbm, geo_hbm, ctab_hbm, keep_hbm,
                     vtab_v, fidx_v, fnrm_v, geo_v, ctab_loc, keep_v):
    wid = lax.axis_index("s") * NC + lax.axis_index("c")
    base = wid * F_PER_W
    pltpu.sync_copy(vtab_hbm, vtab_v)
    for k in range(3):
        pltpu.sync_copy(fidx_hbm.at[pl.ds(k * FP + base, F_PER_W)],
                        fidx_v.at[pl.ds(k * F_PER_W, F_PER_W)])
        pltpu.sync_copy(fnrm_hbm.at[pl.ds(k * FP + base, F_PER_W)],
                        fnrm_v.at[pl.ds(k * F_PER_W, F_PER_W)])

    lane = lax.iota(jnp.int32, L)
    for j in range(F_PER_W // L):
        i0 = fidx_v[pl.ds(j * L, L)]
        i1 = fidx_v[pl.ds(F_PER_W + j * L, L)]
        i2 = fidx_v[pl.ds(2 * F_PER_W + j * L, L)]

        def ga(row, idx):
            return plsc.load_gather(vtab_v, [idx + row * NVP])

        x0, x1, x2 = ga(0, i0), ga(0, i1), ga(0, i2)
        y0, y1, y2 = ga(1, i0), ga(1, i1), ga(1, i2)
        z0, z1, z2 = ga(2, i0), ga(2, i1), ga(2, i2)
        ndot = (ga(6, i0) * fnrm_v[pl.ds(j * L, L)]
                + ga(7, i0) * fnrm_v[pl.ds(F_PER_W + j * L, L)]
                + ga(8, i0) * fnrm_v[pl.ds(2 * F_PER_W + j * L, L)])
        valid = (ndot < 0.0) & (jnp.minimum(z0, jnp.minimum(z1, z2)) > 0.0)

        det = (y1 - y2) * (x0 - x2) + (x2 - x1) * (y0 - y2)
        det = jnp.sign(det) * jnp.maximum(jnp.abs(det), FTINY)
        inv = 1.0 / det
        l0x = (y1 - y2) * inv
        l0y = (x2 - x1) * inv
        l0c = -l0x * x2 - l0y * y2
        l1x = (y2 - y0) * inv
        l1y = (x0 - x2) * inv
        l1c = -l1x * x2 - l1y * y2
        l2x = -l0x - l1x
        l2y = -l0y - l1y
        l2c = 1.0 - l0c - l1c
        dx = z0 * l0x + z1 * l1x + z2 * l2x
        dy = z0 * l0y + z1 * l1y + z2 * l2y
        dc = z0 * l0c + z1 * l1c + z2 * l2c
        l0c_eff = jnp.where(valid, l0c, jnp.float32(-jnp.inf))

        geo = [l0x, l0y, l0c_eff, l1x, l1y, l1c, l2x, l2y, l2c, dx, dy, dc]
        for k, v in enumerate(geo):
            geo_v[pl.ds(k * F_PER_W + j * L, L)] = v
        for n, crow in enumerate((3, 4, 5)):
            c0, c1, c2 = ga(crow, i0), ga(crow, i1), ga(crow, i2)
            cx = c0 * l0x + c1 * l1x + c2 * l2x
            cy = c0 * l0y + c1 * l1y + c2 * l2y
            cc = c0 * l0c + c1 * l1c + c2 * l2c
            ctab_loc[pl.ds((3 * n) * F_PER_W + j * L, L)] = cx
            ctab_loc[pl.ds((3 * n + 1) * F_PER_W + j * L, L)] = cy
            ctab_loc[pl.ds((3 * n + 2) * F_PER_W + j * L, L)] = cc

        bound = (jnp.abs(dx) + jnp.abs(dy)) * 224.0 + jnp.abs(dc)
        danger = (bound >= CULL_TH) | (bound != bound)
        gid = base + j * L + lane
        keep = (valid | danger) & (gid < NF)
        keep_v[pl.ds(j * L, L)] = keep.astype(jnp.int32)

    for k in range(12):
        pltpu.sync_copy(geo_v.at[pl.ds(k * F_PER_W, F_PER_W)],
                        geo_hbm.at[pl.ds(k * FP + base, F_PER_W)])
    for k in range(9):
        pltpu.sync_copy(ctab_loc.at[pl.ds(k * F_PER_W, F_PER_W)],
                        ctab_hbm.at[pl.ds(k * FP + base, F_PER_W)])
    pltpu.sync_copy(keep_v, keep_hbm.at[pl.ds(base, F_PER_W)])


def _face_table(vtab, fidx, fnrm):
    mesh = plsc.VectorSubcoreMesh(core_axis_name="c", subcore_axis_name="s")
    kfn = functools.partial(
        pl.kernel,
        out_type=[
            jax.ShapeDtypeStruct((12 * FP,), jnp.float32),
            jax.ShapeDtypeStruct((9 * FP,), jnp.float32),
            jax.ShapeDtypeStruct((FP,), jnp.int32),
        ],
        mesh=mesh,
        scratch_types=[
            pltpu.VMEM((9 * NVP,), jnp.float32),
            pltpu.VMEM((3 * F_PER_W,), jnp.int32),
            pltpu.VMEM((3 * F_PER_W,), jnp.float32),
            pltpu.VMEM((12 * F_PER_W,), jnp.float32),
            pltpu.VMEM((9 * F_PER_W,), jnp.float32),
            pltpu.VMEM((F_PER_W,), jnp.int32),
        ],
        compiler_params=pltpu.CompilerParams(needs_layout_passes=False),
    )(_face_table_body)
    return kfn(vtab, fidx, fnrm)


def _compact_body(geo_hbm, perm_hbm, geoc_hbm, geo_v, perm_v, out_v):
    wid = lax.axis_index("s") * NC + lax.axis_index("c")
    base = wid * F_PER_W
    pltpu.sync_copy(geo_hbm, geo_v)
    pltpu.sync_copy(perm_hbm.at[pl.ds(base, F_PER_W)], perm_v)

    for j in range(F_PER_W // L):
        pv = perm_v[pl.ds(j * L, L)]
        for k in range(12):
            g = plsc.load_gather(geo_v, [pv + k * FP])
            out_v[pl.ds(k * F_PER_W + j * L, L)] = g

    for k in range(12):
        pltpu.sync_copy(out_v.at[pl.ds(k * F_PER_W, F_PER_W)],
                        geoc_hbm.at[pl.ds(k * FP + base, F_PER_W)])


def _compact(geo, perm):
    mesh = plsc.VectorSubcoreMesh(core_axis_name="c", subcore_axis_name="s")
    kfn = functools.partial(
        pl.kernel,
        out_type=jax.ShapeDtypeStruct((12 * FP,), jnp.float32),
        mesh=mesh,
        scratch_types=[
            pltpu.VMEM((12 * FP,), jnp.float32),
            pltpu.VMEM((F_PER_W,), jnp.int32),
            pltpu.VMEM((12 * F_PER_W,), jnp.float32),
        ],
        compiler_params=pltpu.CompilerParams(needs_layout_passes=False),
    )(_compact_body)
    return kfn(geo, perm)


def _zbuf_body(nc_ref, geo_ref, perm_ref, mask_ref, idx_ref, bestS, bidxS):
    QS = 56
    xv0 = lax.broadcasted_iota(jnp.int32, (QS, 128), 0).astype(jnp.float32)
    nchunk = nc_ref[0]

    def row_body(r, _):
        yf = r.astype(jnp.float32)

        for q in range(4):
            xv = xv0 + jnp.float32(q * QS)

            def chunk_body(c, carry):
                best, bidx = carry
                cs = pl.ds(pl.multiple_of(c * 128, 128), 128)

                def t(k):
                    return geo_ref[k:k + 1, cs]

                ids = perm_ref[0:1, cs]
                l0 = xv * t(0) + (yf * t(1) + t(2))
                l1 = xv * t(3) + (yf * t(4) + t(5))
                l2 = xv * t(6) + (yf * t(7) + t(8))
                m = jnp.minimum(jnp.minimum(l0, l1), l2) >= 0.0
                dd = xv * t(9) + (yf * t(10) + t(11))
                dd = dd + jnp.where(m, 0.0, INF_VALUE)
                dd = jnp.where(dd != dd, INF_VALUE, dd)
                upd = dd < best
                best = jnp.minimum(best, dd)
                bidx = jnp.where(upd, jnp.broadcast_to(ids, (QS, 128)), bidx)
                return best, bidx

            best = jnp.full((QS, 128), INF_VALUE, jnp.float32)
            bidx = jnp.zeros((QS, 128), jnp.int32)
            best, bidx = lax.fori_loop(0, nchunk, chunk_body, (best, bidx))
            bestS[q * QS:(q + 1) * QS, :] = best
            bidxS[q * QS:(q + 1) * QS, :] = bidx

        ball = bestS[:, :]
        iall = bidxS[:, :]
        gm = jnp.min(ball, axis=1, keepdims=True)
        eq = ball == gm
        cand = jnp.where(eq, iall, jnp.int32(2 ** 30))
        arg = jnp.min(cand, axis=1, keepdims=True)
        vis = gm < LOWER_INF
        off = pl.multiple_of(r * W, 8)
        mask_ref[pl.ds(off, W), 0:1] = vis.astype(jnp.float32)
        idx_ref[pl.ds(off, W), 0:1] = jnp.where(vis, arg, 0)
        return 0

    lax.fori_loop(0, H, row_body, 0)


def _zbuffer(nchunks, geo, perm):
    return pl.pallas_call(
        _zbuf_body,
        in_specs=[
            pl.BlockSpec(memory_space=pltpu.SMEM),
            pl.BlockSpec(memory_space=pltpu.VMEM),
            pl.BlockSpec(memory_space=pltpu.VMEM),
        ],
        out_shape=[
            jax.ShapeDtypeStruct((P, 8), jnp.float32),
            jax.ShapeDtypeStruct((P, 8), jnp.int32),
        ],
        scratch_shapes=[
            pltpu.VMEM((W, 128), jnp.float32),
            pltpu.VMEM((W, 128), jnp.int32),
        ],
    )(nchunks, geo, perm)


def _compose_body(ctab_hbm, idx_hbm, mask_hbm, xg_hbm, yg_hbm, img_hbm,
                  ctab_v, idx_v, mask_v, xg_v, yg_v, out_v):
    wid = lax.axis_index("s") * NC + lax.axis_index("c")
    base = wid * PX_PER_W
    pltpu.sync_copy(ctab_hbm, ctab_v)
    pltpu.sync_copy(idx_hbm.at[pl.ds(base, PX_PER_W)], idx_v)
    pltpu.sync_copy(mask_hbm.at[pl.ds(base, PX_PER_W)], mask_v)
    pltpu.sync_copy(xg_hbm.at[pl.ds(base, PX_PER_W)], xg_v)
    pltpu.sync_copy(yg_hbm.at[pl.ds(base, PX_PER_W)], yg_v)

    for j in range(PX_PER_W // L):
        sl = pl.ds(j * L, L)
        iv = idx_v[sl]
        mv = mask_v[sl]
        xv = xg_v[sl]
        yv = yg_v[sl]
        for c in range(3):
            cx = plsc.load_gather(ctab_v, [iv + (3 * c) * FP])
            cy = plsc.load_gather(ctab_v, [iv + (3 * c + 1) * FP])
            cc = plsc.load_gather(ctab_v, [iv + (3 * c + 2) * FP])
            out_v[pl.ds(c * PX_PER_W + j * L, L)] = mv * (cx * xv + cy * yv + cc)

    for c in range(3):
        pltpu.sync_copy(out_v.at[pl.ds(c * PX_PER_W, PX_PER_W)],
                        img_hbm.at[pl.ds(c * P + base, PX_PER_W)])


def _compose(ctab, idx, mask, xg, yg):
    mesh = plsc.VectorSubcoreMesh(core_axis_name="c", subcore_axis_name="s")
    kfn = functools.partial(
        pl.kernel,
        out_type=jax.ShapeDtypeStruct((3 * P,), jnp.float32),
        mesh=mesh,
        scratch_types=[
            pltpu.VMEM((9 * FP,), jnp.float32),
            pltpu.VMEM((PX_PER_W,), jnp.int32),
            pltpu.VMEM((PX_PER_W,), jnp.float32),
            pltpu.VMEM((PX_PER_W,), jnp.float32),
            pltpu.VMEM((PX_PER_W,), jnp.float32),
            pltpu.VMEM((3 * PX_PER_W,), jnp.float32),
        ],
        compiler_params=pltpu.CompilerParams(needs_layout_passes=False),
    )(_compose_body)
    return kfn(ctab, idx, mask, xg, yg)


def kernel(pt_2d, color, pt_3d, normal, R, T, face):
    q = pt_3d[0] + R[0].T @ T[0]
    vtab = jnp.concatenate([pt_2d[0], color[0], q], axis=0)
    vtab = jnp.pad(vtab, ((0, 0), (0, NVP - NV))).reshape(-1)
    fidx = jnp.pad(face, ((0, 0), (0, FP - NF))).reshape(-1)
    fnrm = jnp.pad(normal[0], ((0, 0), (0, FP - NF))).reshape(-1)

    geo_flat, ctab, keep = _face_table(vtab, fidx, fnrm)

    perm = jnp.nonzero(keep, size=FP, fill_value=NF)[0].astype(jnp.int32)
    nkeep = jnp.sum(keep)
    nchunks = ((nkeep + 127) // 128).astype(jnp.int32).reshape(1)

    geoc = _compact(geo_flat, perm).reshape(12, FP)
    mask8, idx8 = _zbuffer(nchunks, geoc, perm.reshape(1, FP))
    mask = mask8[:, 0]
    idx = idx8[:, 0]

    xg = jnp.tile(jnp.arange(W, dtype=jnp.float32), H)
    yg = jnp.repeat(jnp.arange(H, dtype=jnp.float32), W)
    img = _compose(ctab, idx, mask, xg, yg)
    image = img.reshape(1, 3, H, W)
    return image, mask.reshape(1, H, W)


# speedup vs baseline: 5.5920x; 1.0560x over previous
import functools

import jax
import jax.numpy as jnp
import numpy as np
from jax import lax
from jax.experimental import pallas as pl
from jax.experimental.pallas import tpu as pltpu
from jax.experimental.pallas import tpu_sc as plsc

FTINY = float(np.finfo(np.float32).tiny) * 1e3
INF_VALUE = float(np.finfo(np.float32).max) * 1e-3
LOWER_INF = float(np.finfo(np.float32).max) * 1e-4
CULL_TH = 2.8e35

H = 224
W = 224
NF = 4000
NV = 2100
FP = 4096
NVP = 2112
P = H * W

NC = 2
NS = 16
NWORK = NC * NS
L = 16

F_PER_W = FP // NWORK
PX_PER_W = P // NWORK
NCHUNK = FP // 128


def _face_table_body(vtab_hbm, fidx_hbm, fnrm_hbm, geo_hbm, ctab_hbm, keep_hbm,
                     vtab_v, fidx_v, fnrm_v, geo_v, ctab_loc, keep_v):
    wid = lax.axis_index("s") * NC + lax.axis_index("c")
    base = wid * F_PER_W
    pltpu.sync_copy(vtab_hbm, vtab_v)
    for k in range(3):
        pltpu.sync_copy(fidx_hbm.at[pl.ds(k * FP + base, F_PER_W)],
                        fidx_v.at[pl.ds(k * F_PER_W, F_PER_W)])
        pltpu.sync_copy(fnrm_hbm.at[pl.ds(k * FP + base, F_PER_W)],
                        fnrm_v.at[pl.ds(k * F_PER_W, F_PER_W)])

    lane = lax.iota(jnp.int32, L)
    for j in range(F_PER_W // L):
        i0 = fidx_v[pl.ds(j * L, L)]
        i1 = fidx_v[pl.ds(F_PER_W + j * L, L)]
        i2 = fidx_v[pl.ds(2 * F_PER_W + j * L, L)]

        def ga(row, idx):
            return plsc.load_gather(vtab_v, [idx + row * NVP])

        x0, x1, x2 = ga(0, i0), ga(0, i1), ga(0, i2)
        y0, y1, y2 = ga(1, i0), ga(1, i1), ga(1, i2)
        z0, z1, z2 = ga(2, i0), ga(2, i1), ga(2, i2)
        ndot = (ga(6, i0) * fnrm_v[pl.ds(j * L, L)]
                + ga(7, i0) * fnrm_v[pl.ds(F_PER_W + j * L, L)]
                + ga(8, i0) * fnrm_v[pl.ds(2 * F_PER_W + j * L, L)])
        valid = (ndot < 0.0) & (jnp.minimum(z0, jnp.minimum(z1, z2)) > 0.0)

        det = (y1 - y2) * (x0 - x2) + (x2 - x1) * (y0 - y2)
        det = jnp.sign(det) * jnp.maximum(jnp.abs(det), FTINY)
        inv = 1.0 / det
        l0x = (y1 - y2) * inv
        l0y = (x2 - x1) * inv
        l0c = -l0x * x2 - l0y * y2
        l1x = (y2 - y0) * inv
        l1y = (x0 - x2) * inv
        l1c = -l1x * x2 - l1y * y2
        l2x = -l0x - l1x
        l2y = -l0y - l1y
        l2c = 1.0 - l0c - l1c
        dx = z0 * l0x + z1 * l1x + z2 * l2x
        dy = z0 * l0y + z1 * l1y + z2 * l2y
        dc = z0 * l0c + z1 * l1c + z2 * l2c
        l0c_eff = jnp.where(valid, l0c, jnp.float32(-jnp.inf))

        geo = [l0x, l0y, l0c_eff, l1x, l1y, l1c, l2x, l2y, l2c, dx, dy, dc]
        for k, v in enumerate(geo):
            geo_v[pl.ds(k * F_PER_W + j * L, L)] = v
        for n, crow in enumerate((3, 4, 5)):
            c0, c1, c2 = ga(crow, i0), ga(crow, i1), ga(crow, i2)
            cx = c0 * l0x + c1 * l1x + c2 * l2x
            cy = c0 * l0y + c1 * l1y + c2 * l2y
            cc = c0 * l0c + c1 * l1c + c2 * l2c
            ctab_loc[pl.ds((3 * n) * F_PER_W + j * L, L)] = cx
            ctab_loc[pl.ds((3 * n + 1) * F_PER_W + j * L, L)] = cy
            ctab_loc[pl.ds((3 * n + 2) * F_PER_W + j * L, L)] = cc

        bound = (jnp.abs(dx) + jnp.abs(dy)) * 224.0 + jnp.abs(dc)
        danger = (bound >= CULL_TH) | (bound != bound)
        gid = base + j * L + lane
        keep = (valid | danger) & (gid < NF)
        keep_v[pl.ds(j * L, L)] = keep.astype(jnp.int32)

    for k in range(12):
        pltpu.sync_copy(geo_v.at[pl.ds(k * F_PER_W, F_PER_W)],
                        geo_hbm.at[pl.ds(k * FP + base, F_PER_W)])
    for k in range(9):
        pltpu.sync_copy(ctab_loc.at[pl.ds(k * F_PER_W, F_PER_W)],
                        ctab_hbm.at[pl.ds(k * FP + base, F_PER_W)])
    pltpu.sync_copy(keep_v, keep_hbm.at[pl.ds(base, F_PER_W)])


def _face_table(vtab, fidx, fnrm):
    mesh = plsc.VectorSubcoreMesh(core_axis_name="c", subcore_axis_name="s")
    kfn = functools.partial(
        pl.kernel,
        out_type=[
            jax.ShapeDtypeStruct((12 * FP,), jnp.float32),
            jax.ShapeDtypeStruct((9 * FP,), jnp.float32),
            jax.ShapeDtypeStruct((FP,), jnp.int32),
        ],
        mesh=mesh,
        scratch_types=[
            pltpu.VMEM((9 * NVP,), jnp.float32),
            pltpu.VMEM((3 * F_PER_W,), jnp.int32),
            pltpu.VMEM((3 * F_PER_W,), jnp.float32),
            pltpu.VMEM((12 * F_PER_W,), jnp.float32),
            pltpu.VMEM((9 * F_PER_W,), jnp.float32),
            pltpu.VMEM((F_PER_W,), jnp.int32),
        ],
        compiler_params=pltpu.CompilerParams(needs_layout_passes=False),
    )(_face_table_body)
    return kfn(vtab, fidx, fnrm)


def _compact_body(geo_hbm, perm_hbm, geoc_hbm, geo_v, perm_v, out_v):
    wid = lax.axis_index("s") * NC + lax.axis_index("c")
    base = wid * F_PER_W
    pltpu.sync_copy(geo_hbm, geo_v)
    pltpu.sync_copy(perm_hbm.at[pl.ds(base, F_PER_W)], perm_v)

    for j in range(F_PER_W // L):
        pv = perm_v[pl.ds(j * L, L)]
        for k in range(12):
            g = plsc.load_gather(geo_v, [pv + k * FP])
            out_v[pl.ds(k * F_PER_W + j * L, L)] = g

    for k in range(12):
        pltpu.sync_copy(out_v.at[pl.ds(k * F_PER_W, F_PER_W)],
                        geoc_hbm.at[pl.ds(k * FP + base, F_PER_W)])


def _compact(geo, perm):
    mesh = plsc.VectorSubcoreMesh(core_axis_name="c", subcore_axis_name="s")
    kfn = functools.partial(
        pl.kernel,
        out_type=jax.ShapeDtypeStruct((12 * FP,), jnp.float32),
        mesh=mesh,
        scratch_types=[
            pltpu.VMEM((12 * FP,), jnp.float32),
            pltpu.VMEM((F_PER_W,), jnp.int32),
            pltpu.VMEM((12 * F_PER_W,), jnp.float32),
        ],
        compiler_params=pltpu.CompilerParams(needs_layout_passes=False),
    )(_compact_body)
    return kfn(geo, perm)


def _zbuf_body(nc_ref, geo_ref, perm_ref, mask_ref, idx_ref, bestS, bidxS):
    QS = 56
    xv0 = lax.broadcasted_iota(jnp.int32, (QS, 128), 0).astype(jnp.float32)
    nchunk = nc_ref[0]

    def row_body(r, _):
        yf = r.astype(jnp.float32)

        for q in range(4):
            xv = xv0 + jnp.float32(q * QS)

            def chunk_body(c, carry):
                best, bidx = carry
                for h in range(2):
                    cs = pl.ds(pl.multiple_of(c * 256 + h * 128, 128), 128)

                    def t(k):
                        return geo_ref[k:k + 1, cs]

                    ids = perm_ref[0:1, cs]
                    l0 = xv * t(0) + (yf * t(1) + t(2))
                    l1 = xv * t(3) + (yf * t(4) + t(5))
                    l2 = xv * t(6) + (yf * t(7) + t(8))
                    m = jnp.minimum(jnp.minimum(l0, l1), l2) >= 0.0
                    dd = xv * t(9) + (yf * t(10) + t(11))
                    dd = dd + jnp.where(m, 0.0, INF_VALUE)
                    dd = jnp.where(dd != dd, INF_VALUE, dd)
                    upd = dd < best
                    best = jnp.minimum(best, dd)
                    bidx = jnp.where(upd, jnp.broadcast_to(ids, (QS, 128)),
                                     bidx)
                return best, bidx

            best = jnp.full((QS, 128), INF_VALUE, jnp.float32)
            bidx = jnp.zeros((QS, 128), jnp.int32)
            best, bidx = lax.fori_loop(0, nchunk, chunk_body, (best, bidx))
            bestS[q * QS:(q + 1) * QS, :] = best
            bidxS[q * QS:(q + 1) * QS, :] = bidx

        ball = bestS[:, :]
        iall = bidxS[:, :]
        gm = jnp.min(ball, axis=1, keepdims=True)
        eq = ball == gm
        cand = jnp.where(eq, iall, jnp.int32(2 ** 30))
        arg = jnp.min(cand, axis=1, keepdims=True)
        vis = gm < LOWER_INF
        off = pl.multiple_of(r * W, 8)
        mask_ref[pl.ds(off, W), 0:1] = vis.astype(jnp.float32)
        idx_ref[pl.ds(off, W), 0:1] = jnp.where(vis, arg, 0)
        return 0

    lax.fori_loop(0, H, row_body, 0)


def _zbuffer(nchunks, geo, perm):
    return pl.pallas_call(
        _zbuf_body,
        in_specs=[
            pl.BlockSpec(memory_space=pltpu.SMEM),
            pl.BlockSpec(memory_space=pltpu.VMEM),
            pl.BlockSpec(memory_space=pltpu.VMEM),
        ],
        out_shape=[
            jax.ShapeDtypeStruct((P, 8), jnp.float32),
            jax.ShapeDtypeStruct((P, 8), jnp.int32),
        ],
        scratch_shapes=[
            pltpu.VMEM((W, 128), jnp.float32),
            pltpu.VMEM((W, 128), jnp.int32),
        ],
    )(nchunks, geo, perm)


def _compose_body(ctab_hbm, idx_hbm, mask_hbm, xg_hbm, yg_hbm, img_hbm,
                  ctab_v, idx_v, mask_v, xg_v, yg_v, out_v):
    wid = lax.axis_index("s") * NC + lax.axis_index("c")
    base = wid * PX_PER_W
    pltpu.sync_copy(ctab_hbm, ctab_v)
    pltpu.sync_copy(idx_hbm.at[pl.ds(base, PX_PER_W)], idx_v)
    pltpu.sync_copy(mask_hbm.at[pl.ds(base, PX_PER_W)], mask_v)
    pltpu.sync_copy(xg_hbm.at[pl.ds(base, PX_PER_W)], xg_v)
    pltpu.sync_copy(yg_hbm.at[pl.ds(base, PX_PER_W)], yg_v)

    for j in range(PX_PER_W // L):
        sl = pl.ds(j * L, L)
        iv = idx_v[sl]
        mv = mask_v[sl]
        xv = xg_v[sl]
        yv = yg_v[sl]
        for c in range(3):
            cx = plsc.load_gather(ctab_v, [iv + (3 * c) * FP])
            cy = plsc.load_gather(ctab_v, [iv + (3 * c + 1) * FP])
            cc = plsc.load_gather(ctab_v, [iv + (3 * c + 2) * FP])
            out_v[pl.ds(c * PX_PER_W + j * L, L)] = mv * (cx * xv + cy * yv + cc)

    for c in range(3):
        pltpu.sync_copy(out_v.at[pl.ds(c * PX_PER_W, PX_PER_W)],
                        img_hbm.at[pl.ds(c * P + base, PX_PER_W)])


def _compose(ctab, idx, mask, xg, yg):
    mesh = plsc.VectorSubcoreMesh(core_axis_name="c", subcore_axis_name="s")
    kfn = functools.partial(
        pl.kernel,
        out_type=jax.ShapeDtypeStruct((3 * P,), jnp.float32),
        mesh=mesh,
        scratch_types=[
            pltpu.VMEM((9 * FP,), jnp.float32),
            pltpu.VMEM((PX_PER_W,), jnp.int32),
            pltpu.VMEM((PX_PER_W,), jnp.float32),
            pltpu.VMEM((PX_PER_W,), jnp.float32),
            pltpu.VMEM((PX_PER_W,), jnp.float32),
            pltpu.VMEM((3 * PX_PER_W,), jnp.float32),
        ],
        compiler_params=pltpu.CompilerParams(needs_layout_passes=False),
    )(_compose_body)
    return kfn(ctab, idx, mask, xg, yg)


def kernel(pt_2d, color, pt_3d, normal, R, T, face):
    q = pt_3d[0] + R[0].T @ T[0]
    vtab = jnp.concatenate([pt_2d[0], color[0], q], axis=0)
    vtab = jnp.pad(vtab, ((0, 0), (0, NVP - NV))).reshape(-1)
    fidx = jnp.pad(face, ((0, 0), (0, FP - NF))).reshape(-1)
    fnrm = jnp.pad(normal[0], ((0, 0), (0, FP - NF))).reshape(-1)

    geo_flat, ctab, keep = _face_table(vtab, fidx, fnrm)

    perm = jnp.nonzero(keep, size=FP, fill_value=NF)[0].astype(jnp.int32)
    nkeep = jnp.sum(keep)
    nchunks = ((nkeep + 255) // 256).astype(jnp.int32).reshape(1)

    geoc = _compact(geo_flat, perm).reshape(12, FP)
    mask8, idx8 = _zbuffer(nchunks, geoc, perm.reshape(1, FP))
    mask = mask8[:, 0]
    idx = idx8[:, 0]

    xg = jnp.tile(jnp.arange(W, dtype=jnp.float32), H)
    yg = jnp.repeat(jnp.arange(H, dtype=jnp.float32), W)
    img = _compose(ctab, idx, mask, xg, yg)
    image = img.reshape(1, 3, H, W)
    return image, mask.reshape(1, H, W)
